# Initial kernel scaffold; baseline (speedup 1.0000x reference)
import functools

import jax
import jax.numpy as jnp
from jax import lax
from jax.experimental import pallas as pl
from jax.experimental.pallas import tpu as pltpu
from jax.experimental.pallas import tpu_sc as plsc

N = 10000
E = 320000
D = 128

NC = 2
NS = 16
NW = NC * NS
EPW = E // NW
C = 100
CH = EPW // C
RPS = N // NS

_MESH = plsc.VectorSubcoreMesh(core_axis_name="c", subcore_axis_name="s")


def _sc_degree(dst3, ones_c16, zeros16):
    @functools.partial(
        pl.kernel,
        out_type=jax.ShapeDtypeStruct((NC, N, 16), jnp.float32),
        mesh=_MESH,
        scratch_types=[
            pltpu.VMEM((CH, C), jnp.int32),
            pltpu.VMEM((C, 16), jnp.float32),
            pltpu.VMEM_SHARED((N, 16), jnp.float32),
        ],
    )
    def k(dst_hbm, ones_hbm, zeros_hbm, out_hbm, idx_v, ones_v, acc_sh):
        c = lax.axis_index("c")
        s = lax.axis_index("s")
        wid = c * NS + s
        pltpu.sync_copy(zeros_hbm, acc_sh.at[pl.ds(s * RPS, RPS)])
        pltpu.sync_copy(ones_hbm, ones_v)
        pltpu.sync_copy(dst_hbm.at[wid], idx_v)
        plsc.subcore_barrier()

        @pl.loop(0, CH)
        def _(j):
            pltpu.sync_copy(ones_v, acc_sh.at[idx_v.at[j]], add=True)

        plsc.subcore_barrier()
        pltpu.sync_copy(
            acc_sh.at[pl.ds(s * RPS, RPS)],
            out_hbm.at[c, pl.ds(s * RPS, RPS)],
        )

    return k(dst3, ones_c16, zeros16)


def _sc_scatter(hn, src3, dst3, zeros128):
    @functools.partial(
        pl.kernel,
        out_type=jax.ShapeDtypeStruct((NC, N, D), jnp.float32),
        mesh=_MESH,
        scratch_types=[
            pltpu.VMEM((CH, C), jnp.int32),
            pltpu.VMEM((CH, C), jnp.int32),
            pltpu.VMEM((C, D), jnp.float32),
            pltpu.VMEM_SHARED((N, D), jnp.float32),
        ],
    )
    def k(hn_hbm, src_hbm, dst_hbm, zeros_hbm, out_hbm,
          src_v, dst_v, rows_v, acc_sh):
        c = lax.axis_index("c")
        s = lax.axis_index("s")
        wid = c * NS + s
        pltpu.sync_copy(zeros_hbm, acc_sh.at[pl.ds(s * RPS, RPS)])
        pltpu.sync_copy(src_hbm.at[wid], src_v)
        pltpu.sync_copy(dst_hbm.at[wid], dst_v)
        plsc.subcore_barrier()

        @pl.loop(0, CH)
        def _(j):
            pltpu.sync_copy(hn_hbm.at[src_v.at[j]], rows_v)
            pltpu.sync_copy(rows_v, acc_sh.at[dst_v.at[j]], add=True)

        plsc.subcore_barrier()
        pltpu.sync_copy(
            acc_sh.at[pl.ds(s * RPS, RPS)],
            out_hbm.at[c, pl.ds(s * RPS, RPS)],
        )

    return k(hn, src3, dst3, zeros128)


_BR = 2000


def _tc_mm1(x, W1):
    def body(x_ref, w_ref, o_ref):
        o_ref[...] = jnp.dot(x_ref[...], w_ref[...],
                             preferred_element_type=jnp.float32)

    return pl.pallas_call(
        body,
        grid=(N // _BR,),
        in_specs=[
            pl.BlockSpec((_BR, D), lambda i: (i, 0)),
            pl.BlockSpec((D, D), lambda i: (0, 0)),
        ],
        out_specs=pl.BlockSpec((_BR, D), lambda i: (i, 0)),
        out_shape=jax.ShapeDtypeStruct((N, D), jnp.float32),
    )(x, W1)


def _tc_scale(h1, dega, degb):
    def body(h_ref, da_ref, db_ref, o_ref):
        dinv = lax.rsqrt(da_ref[:, :1] + db_ref[:, :1] + 1.0)
        o_ref[...] = h_ref[...] * dinv

    return pl.pallas_call(
        body,
        grid=(N // _BR,),
        in_specs=[
            pl.BlockSpec((_BR, D), lambda i: (i, 0)),
            pl.BlockSpec((_BR, 16), lambda i: (i, 0)),
            pl.BlockSpec((_BR, 16), lambda i: (i, 0)),
        ],
        out_specs=pl.BlockSpec((_BR, D), lambda i: (i, 0)),
        out_shape=jax.ShapeDtypeStruct((N, D), jnp.float32),
    )(h1, dega, degb)


def _tc_mid(acca, accb, hn1, dega, degb, b1r, W2):
    def body(aa_ref, ab_ref, hn_ref, da_ref, db_ref, b_ref, w_ref, o_ref):
        dinv = lax.rsqrt(da_ref[:, :1] + db_ref[:, :1] + 1.0)
        s = aa_ref[...] + ab_ref[...] + hn_ref[...]
        o1 = jnp.maximum(dinv * s + b_ref[...], 0.0)
        h2 = jnp.dot(o1, w_ref[...], preferred_element_type=jnp.float32)
        o_ref[...] = h2 * dinv

    return pl.pallas_call(
        body,
        grid=(N // _BR,),
        in_specs=[
            pl.BlockSpec((_BR, D), lambda i: (i, 0)),
            pl.BlockSpec((_BR, D), lambda i: (i, 0)),
            pl.BlockSpec((_BR, D), lambda i: (i, 0)),
            pl.BlockSpec((_BR, 16), lambda i: (i, 0)),
            pl.BlockSpec((_BR, 16), lambda i: (i, 0)),
            pl.BlockSpec((1, D), lambda i: (0, 0)),
            pl.BlockSpec((D, D), lambda i: (0, 0)),
        ],
        out_specs=pl.BlockSpec((_BR, D), lambda i: (i, 0)),
        out_shape=jax.ShapeDtypeStruct((N, D), jnp.float32),
    )(acca, accb, hn1, dega, degb, b1r, W2)


def _tc_fin(acca, accb, hn2, dega, degb):
    def body(aa_ref, ab_ref, hn_ref, da_ref, db_ref, o_ref):
        dinv = lax.rsqrt(da_ref[:, :1] + db_ref[:, :1] + 1.0)
        z = dinv * (aa_ref[...] + ab_ref[...] + hn_ref[...])
        m = jnp.max(z, axis=1, keepdims=True)
        lse = jnp.log(jnp.sum(jnp.exp(z - m), axis=1, keepdims=True))
        o_ref[...] = z - m - lse

    return pl.pallas_call(
        body,
        grid=(N // _BR,),
        in_specs=[
            pl.BlockSpec((_BR, D), lambda i: (i, 0)),
            pl.BlockSpec((_BR, D), lambda i: (i, 0)),
            pl.BlockSpec((_BR, D), lambda i: (i, 0)),
            pl.BlockSpec((_BR, 16), lambda i: (i, 0)),
            pl.BlockSpec((_BR, 16), lambda i: (i, 0)),
        ],
        out_specs=pl.BlockSpec((_BR, D), lambda i: (i, 0)),
        out_shape=jax.ShapeDtypeStruct((N, D), jnp.float32),
    )(acca, accb, hn2, dega, degb)


def kernel(x, edge_index, W1, b1, W2):
    src3 = edge_index[0].reshape(NW, CH, C)
    dst3 = edge_index[1].reshape(NW, CH, C)
    ones_c16 = jnp.ones((C, 16), jnp.float32)
    zeros16 = jnp.zeros((RPS, 16), jnp.float32)
    zeros128 = jnp.zeros((RPS, D), jnp.float32)

    deg = _sc_degree(dst3, ones_c16, zeros16)
    h1 = _tc_mm1(x, W1)
    hn1 = _tc_scale(h1, deg[0], deg[1])
    acc1 = _sc_scatter(hn1, src3, dst3, zeros128)
    hn2 = _tc_mid(acc1[0], acc1[1], hn1, deg[0], deg[1],
                  b1.reshape(1, D), W2)
    acc2 = _sc_scatter(hn2, src3, dst3, zeros128)
    return _tc_fin(acc2[0], acc2[1], hn2, deg[0], deg[1])


# speedup vs baseline: 18.3451x; 18.3451x over previous
import functools

import jax
import jax.numpy as jnp
from jax import lax
from jax.experimental import pallas as pl
from jax.experimental.pallas import tpu as pltpu
from jax.experimental.pallas import tpu_sc as plsc

N = 10000
E = 320000
D = 128

NC = 2
NS = 16
NW = NC * NS
EPW = E // NW
C = 100
CH = EPW // C
NP = 10240
RPS = NP // NS

def _mesh():
    return plsc.VectorSubcoreMesh(core_axis_name="c", subcore_axis_name="s",
                                  num_cores=NC, num_subcores=NS)


def _sc_degree(dst3, ones_c16, zeros16):
    @functools.partial(
        pl.kernel,
        out_type=jax.ShapeDtypeStruct((NC, NP, D), jnp.float32),
        mesh=_mesh(),
        scratch_types=[
            pltpu.VMEM((CH, C), jnp.int32),
            pltpu.VMEM((C, D), jnp.float32),
            pltpu.VMEM_SHARED((NP, D), jnp.float32),
        ],
    )
    def k(dst_hbm, ones_hbm, zeros_hbm, out_hbm, idx_v, ones_v, acc_sh):
        c = lax.axis_index("c")
        s = lax.axis_index("s")
        wid = c * NS + s
        pltpu.sync_copy(zeros_hbm, acc_sh.at[pl.ds(s * RPS, RPS)])
        pltpu.sync_copy(ones_hbm, ones_v)
        pltpu.sync_copy(dst_hbm.at[wid], idx_v)
        plsc.subcore_barrier()

        @pl.loop(0, CH)
        def _(j):
            pltpu.sync_copy(ones_v, acc_sh.at[idx_v.at[j]], add=True)

        plsc.subcore_barrier()
        pltpu.sync_copy(
            acc_sh.at[pl.ds(s * RPS, RPS)],
            out_hbm.at[c, pl.ds(s * RPS, RPS)],
        )

    return k(dst3, ones_c16, zeros16)


def _sc_scatter(hn, src3, dst3, zeros128):
    @functools.partial(
        pl.kernel,
        out_type=jax.ShapeDtypeStruct((NC, NP, D), jnp.float32),
        mesh=_mesh(),
        scratch_types=[
            pltpu.VMEM((CH, C), jnp.int32),
            pltpu.VMEM((CH, C), jnp.int32),
            pltpu.VMEM((C, D), jnp.float32),
            pltpu.VMEM_SHARED((NP, D), jnp.float32),
        ],
    )
    def k(hn_hbm, src_hbm, dst_hbm, zeros_hbm, out_hbm,
          src_v, dst_v, rows_v, acc_sh):
        c = lax.axis_index("c")
        s = lax.axis_index("s")
        wid = c * NS + s
        pltpu.sync_copy(zeros_hbm, acc_sh.at[pl.ds(s * RPS, RPS)])
        pltpu.sync_copy(src_hbm.at[wid], src_v)
        pltpu.sync_copy(dst_hbm.at[wid], dst_v)
        plsc.subcore_barrier()

        @pl.loop(0, CH)
        def _(j):
            pltpu.sync_copy(hn_hbm.at[src_v.at[j]], rows_v)
            pltpu.sync_copy(rows_v, acc_sh.at[dst_v.at[j]], add=True)

        plsc.subcore_barrier()
        pltpu.sync_copy(
            acc_sh.at[pl.ds(s * RPS, RPS)],
            out_hbm.at[c, pl.ds(s * RPS, RPS)],
        )

    return k(hn, src3, dst3, zeros128)


_BR = 2000


def _tc_mm1(x, W1):
    def body(x_ref, w_ref, o_ref):
        o_ref[...] = jnp.dot(x_ref[...], w_ref[...],
                             preferred_element_type=jnp.float32)

    return pl.pallas_call(
        body,
        grid=(N // _BR,),
        in_specs=[
            pl.BlockSpec((_BR, D), lambda i: (i, 0)),
            pl.BlockSpec((D, D), lambda i: (0, 0)),
        ],
        out_specs=pl.BlockSpec((_BR, D), lambda i: (i, 0)),
        out_shape=jax.ShapeDtypeStruct((N, D), jnp.float32),
    )(x, W1)


def _tc_scale(h1, dega, degb):
    def body(h_ref, da_ref, db_ref, o_ref):
        dinv = lax.rsqrt(da_ref[:, :1] + db_ref[:, :1] + 1.0)
        o_ref[...] = h_ref[...] * dinv

    return pl.pallas_call(
        body,
        grid=(N // _BR,),
        in_specs=[
            pl.BlockSpec((_BR, D), lambda i: (i, 0)),
            pl.BlockSpec((_BR, D), lambda i: (i, 0)),
            pl.BlockSpec((_BR, D), lambda i: (i, 0)),
        ],
        out_specs=pl.BlockSpec((_BR, D), lambda i: (i, 0)),
        out_shape=jax.ShapeDtypeStruct((N, D), jnp.float32),
    )(h1, dega, degb)


def _tc_mid(acca, accb, hn1, dega, degb, b1r, W2):
    def body(aa_ref, ab_ref, hn_ref, da_ref, db_ref, b_ref, w_ref, o_ref):
        dinv = lax.rsqrt(da_ref[:, :1] + db_ref[:, :1] + 1.0)
        s = aa_ref[...] + ab_ref[...] + hn_ref[...]
        o1 = jnp.maximum(dinv * s + b_ref[...], 0.0)
        h2 = jnp.dot(o1, w_ref[...], preferred_element_type=jnp.float32)
        o_ref[...] = h2 * dinv

    return pl.pallas_call(
        body,
        grid=(N // _BR,),
        in_specs=[
            pl.BlockSpec((_BR, D), lambda i: (i, 0)),
            pl.BlockSpec((_BR, D), lambda i: (i, 0)),
            pl.BlockSpec((_BR, D), lambda i: (i, 0)),
            pl.BlockSpec((_BR, D), lambda i: (i, 0)),
            pl.BlockSpec((_BR, D), lambda i: (i, 0)),
            pl.BlockSpec((1, D), lambda i: (0, 0)),
            pl.BlockSpec((D, D), lambda i: (0, 0)),
        ],
        out_specs=pl.BlockSpec((_BR, D), lambda i: (i, 0)),
        out_shape=jax.ShapeDtypeStruct((N, D), jnp.float32),
    )(acca, accb, hn1, dega, degb, b1r, W2)


def _tc_fin(acca, accb, hn2, dega, degb):
    def body(aa_ref, ab_ref, hn_ref, da_ref, db_ref, o_ref):
        dinv = lax.rsqrt(da_ref[:, :1] + db_ref[:, :1] + 1.0)
        z = dinv * (aa_ref[...] + ab_ref[...] + hn_ref[...])
        m = jnp.max(z, axis=1, keepdims=True)
        lse = jnp.log(jnp.sum(jnp.exp(z - m), axis=1, keepdims=True))
        o_ref[...] = z - m - lse

    return pl.pallas_call(
        body,
        grid=(N // _BR,),
        in_specs=[
            pl.BlockSpec((_BR, D), lambda i: (i, 0)),
            pl.BlockSpec((_BR, D), lambda i: (i, 0)),
            pl.BlockSpec((_BR, D), lambda i: (i, 0)),
            pl.BlockSpec((_BR, D), lambda i: (i, 0)),
            pl.BlockSpec((_BR, D), lambda i: (i, 0)),
        ],
        out_specs=pl.BlockSpec((_BR, D), lambda i: (i, 0)),
        out_shape=jax.ShapeDtypeStruct((N, D), jnp.float32),
    )(acca, accb, hn2, dega, degb)


def kernel(x, edge_index, W1, b1, W2):
    src3 = edge_index[0].reshape(NW, CH, C)
    dst3 = edge_index[1].reshape(NW, CH, C)
    ones_c16 = jnp.ones((C, D), jnp.float32)
    zeros128 = jnp.zeros((RPS, D), jnp.float32)

    deg = _sc_degree(dst3, ones_c16, zeros128)
    h1 = _tc_mm1(x, W1)
    hn1 = _tc_scale(h1, deg[0], deg[1])
    acc1 = _sc_scatter(hn1, src3, dst3, zeros128)
    hn2 = _tc_mid(acc1[0], acc1[1], hn1, deg[0], deg[1],
                  b1.reshape(1, D), W2)
    acc2 = _sc_scatter(hn2, src3, dst3, zeros128)
    return _tc_fin(acc2[0], acc2[1], hn2, deg[0], deg[1])


# speedup vs baseline: 22.4180x; 1.2220x over previous
import functools

import jax
import jax.numpy as jnp
from jax import lax
from jax.experimental import pallas as pl
from jax.experimental.pallas import tpu as pltpu
from jax.experimental.pallas import tpu_sc as plsc

N = 10000
E = 320000
D = 128

NC = 2
NS = 16
NW = NC * NS
EPW = E // NW
C = 50
CH = EPW // C
NP = 10240
RPS = NP // NS

def _mesh():
    return plsc.VectorSubcoreMesh(core_axis_name="c", subcore_axis_name="s",
                                  num_cores=NC, num_subcores=NS)


def _sc_degree(dst3, ones_c16, zeros16):
    @functools.partial(
        pl.kernel,
        out_type=jax.ShapeDtypeStruct((NC, NP, D), jnp.float32),
        mesh=_mesh(),
        scratch_types=[
            pltpu.VMEM((CH, C), jnp.int32),
            pltpu.VMEM((C, D), jnp.float32),
            pltpu.VMEM_SHARED((NP, D), jnp.float32),
            pltpu.SemaphoreType.DMA,
        ],
    )
    def k(dst_hbm, ones_hbm, zeros_hbm, out_hbm, idx_v, ones_v, acc_sh, sem):
        c = lax.axis_index("c")
        s = lax.axis_index("s")
        wid = c * NS + s
        pltpu.sync_copy(zeros_hbm, acc_sh.at[pl.ds(s * RPS, RPS)])
        pltpu.sync_copy(ones_hbm, ones_v)
        pltpu.sync_copy(dst_hbm.at[wid], idx_v)
        plsc.subcore_barrier()

        @pl.loop(0, CH)
        def _(j):
            pltpu.async_copy(ones_v, acc_sh.at[idx_v.at[j]], sem, add=True)

        @pl.loop(0, CH)
        def _(j):
            pltpu.make_async_copy(ones_v, acc_sh.at[idx_v.at[j]], sem).wait()

        plsc.subcore_barrier()
        pltpu.sync_copy(
            acc_sh.at[pl.ds(s * RPS, RPS)],
            out_hbm.at[c, pl.ds(s * RPS, RPS)],
        )

    return k(dst3, ones_c16, zeros16)


def _sc_scatter(hn, idx3, zeros128):
    @functools.partial(
        pl.kernel,
        out_type=jax.ShapeDtypeStruct((NC, NP, D), jnp.float32),
        mesh=_mesh(),
        scratch_types=[
            pltpu.VMEM((4, 2, C), jnp.int32),
            pltpu.VMEM((4, C, D), jnp.float32),
            pltpu.VMEM_SHARED((NP, D), jnp.float32),
        ] + [pltpu.SemaphoreType.DMA] * 8,
    )
    def k(hn_hbm, idx_hbm, zeros_hbm, out_hbm,
          idx_v, rows_v, acc_sh, *sems):
        c = lax.axis_index("c")
        s = lax.axis_index("s")
        wid = c * NS + s
        sg = sems[:4]
        si = sems[4:]
        pltpu.sync_copy(zeros_hbm, acc_sh.at[pl.ds(s * RPS, RPS)])
        plsc.subcore_barrier()

        def start_idx(q, b):
            pltpu.async_copy(idx_hbm.at[wid, q], idx_v.at[b], si[b])

        def wait_idx(q, b):
            pltpu.make_async_copy(
                idx_hbm.at[wid, q], idx_v.at[b], si[b]).wait()

        def start_gather(b):
            pltpu.async_copy(hn_hbm.at[idx_v.at[b, 0]], rows_v.at[b], sg[b])

        def wait_gather(b):
            pltpu.make_async_copy(
                hn_hbm.at[idx_v.at[b, 0]], rows_v.at[b], sg[b]).wait()

        def start_scatter(b):
            pltpu.async_copy(rows_v.at[b], acc_sh.at[idx_v.at[b, 1]], sg[b],
                             add=True)

        def wait_scatter(b):
            pltpu.make_async_copy(
                rows_v.at[b], acc_sh.at[idx_v.at[b, 1]], sg[b]).wait()

        for b in range(4):
            start_idx(b, b)
        for b in range(4):
            wait_idx(b, b)
            start_gather(b)

        @pl.loop(0, CH - 4, step=4)
        def _(j):
            for b in range(4):
                wait_gather(b)
                start_scatter(b)
            for b in range(4):
                wait_scatter(b)
                start_idx(j + 4 + b, b)
            for b in range(4):
                wait_idx(j + 4 + b, b)
                start_gather(b)

        for b in range(4):
            wait_gather(b)
            start_scatter(b)
        for b in range(4):
            wait_scatter(b)

        plsc.subcore_barrier()
        pltpu.sync_copy(
            acc_sh.at[pl.ds(s * RPS, RPS)],
            out_hbm.at[c, pl.ds(s * RPS, RPS)],
        )

    return k(hn, idx3, zeros128)


_BR = 2000


def _tc_mm1(x, W1):
    def body(x_ref, w_ref, o_ref):
        o_ref[...] = jnp.dot(x_ref[...], w_ref[...],
                             preferred_element_type=jnp.float32)

    return pl.pallas_call(
        body,
        grid=(N // _BR,),
        in_specs=[
            pl.BlockSpec((_BR, D), lambda i: (i, 0)),
            pl.BlockSpec((D, D), lambda i: (0, 0)),
        ],
        out_specs=pl.BlockSpec((_BR, D), lambda i: (i, 0)),
        out_shape=jax.ShapeDtypeStruct((N, D), jnp.float32),
    )(x, W1)


def _tc_scale(h1, dega, degb):
    def body(h_ref, da_ref, db_ref, o_ref):
        dinv = lax.rsqrt(da_ref[:, :1] + db_ref[:, :1] + 1.0)
        o_ref[...] = h_ref[...] * dinv

    return pl.pallas_call(
        body,
        grid=(N // _BR,),
        in_specs=[
            pl.BlockSpec((_BR, D), lambda i: (i, 0)),
            pl.BlockSpec((_BR, D), lambda i: (i, 0)),
            pl.BlockSpec((_BR, D), lambda i: (i, 0)),
        ],
        out_specs=pl.BlockSpec((_BR, D), lambda i: (i, 0)),
        out_shape=jax.ShapeDtypeStruct((N, D), jnp.float32),
    )(h1, dega, degb)


def _tc_mid(acca, accb, hn1, dega, degb, b1r, W2):
    def body(aa_ref, ab_ref, hn_ref, da_ref, db_ref, b_ref, w_ref, o_ref):
        dinv = lax.rsqrt(da_ref[:, :1] + db_ref[:, :1] + 1.0)
        s = aa_ref[...] + ab_ref[...] + hn_ref[...]
        o1 = jnp.maximum(dinv * s + b_ref[...], 0.0)
        h2 = jnp.dot(o1, w_ref[...], preferred_element_type=jnp.float32)
        o_ref[...] = h2 * dinv

    return pl.pallas_call(
        body,
        grid=(N // _BR,),
        in_specs=[
            pl.BlockSpec((_BR, D), lambda i: (i, 0)),
            pl.BlockSpec((_BR, D), lambda i: (i, 0)),
            pl.BlockSpec((_BR, D), lambda i: (i, 0)),
            pl.BlockSpec((_BR, D), lambda i: (i, 0)),
            pl.BlockSpec((_BR, D), lambda i: (i, 0)),
            pl.BlockSpec((1, D), lambda i: (0, 0)),
            pl.BlockSpec((D, D), lambda i: (0, 0)),
        ],
        out_specs=pl.BlockSpec((_BR, D), lambda i: (i, 0)),
        out_shape=jax.ShapeDtypeStruct((N, D), jnp.float32),
    )(acca, accb, hn1, dega, degb, b1r, W2)


def _tc_fin(acca, accb, hn2, dega, degb):
    def body(aa_ref, ab_ref, hn_ref, da_ref, db_ref, o_ref):
        dinv = lax.rsqrt(da_ref[:, :1] + db_ref[:, :1] + 1.0)
        z = dinv * (aa_ref[...] + ab_ref[...] + hn_ref[...])
        m = jnp.max(z, axis=1, keepdims=True)
        lse = jnp.log(jnp.sum(jnp.exp(z - m), axis=1, keepdims=True))
        o_ref[...] = z - m - lse

    return pl.pallas_call(
        body,
        grid=(N // _BR,),
        in_specs=[
            pl.BlockSpec((_BR, D), lambda i: (i, 0)),
            pl.BlockSpec((_BR, D), lambda i: (i, 0)),
            pl.BlockSpec((_BR, D), lambda i: (i, 0)),
            pl.BlockSpec((_BR, D), lambda i: (i, 0)),
            pl.BlockSpec((_BR, D), lambda i: (i, 0)),
        ],
        out_specs=pl.BlockSpec((_BR, D), lambda i: (i, 0)),
        out_shape=jax.ShapeDtypeStruct((N, D), jnp.float32),
    )(acca, accb, hn2, dega, degb)


def kernel(x, edge_index, W1, b1, W2):
    src3 = edge_index[0].reshape(NW, CH, 1, C)
    dst3 = edge_index[1].reshape(NW, CH, 1, C)
    idx3 = jnp.concatenate([src3, dst3], axis=2)
    ones_c16 = jnp.ones((C, D), jnp.float32)
    zeros128 = jnp.zeros((RPS, D), jnp.float32)

    deg = _sc_degree(dst3.reshape(NW, CH, C), ones_c16, zeros128)
    h1 = _tc_mm1(x, W1)
    hn1 = _tc_scale(h1, deg[0], deg[1])
    acc1 = _sc_scatter(hn1, idx3, zeros128)
    hn2 = _tc_mid(acc1[0], acc1[1], hn1, deg[0], deg[1],
                  b1.reshape(1, D), W2)
    acc2 = _sc_scatter(hn2, idx3, zeros128)
    return _tc_fin(acc2[0], acc2[1], hn2, deg[0], deg[1])
